# baseline (speedup 1.0000x reference)
import functools

import jax
import jax.numpy as jnp
from jax import lax
from jax.experimental import pallas as pl
from jax.experimental.pallas import tpu as pltpu
from jax.experimental.pallas import tpu_sc as plsc

N_PAD = 10240
D = 128
W = 144
NC = 2
NS = 16
NW = NC * NS
CH = 80
EPS = 1e-30


def _dense_body(s_ref, w1_ref, b1_ref, w2_ref, b2_ref, a_ref,
                z_ref, h_ref, b_ref, c_ref):
    i = pl.program_id(0)
    s = s_ref[...]
    dn = (((1,), (1,)), ((), ()))
    z_ref[...] = lax.dot_general(s, w1_ref[...], dn,
                                 preferred_element_type=jnp.float32) + b1_ref[...]
    h = lax.dot_general(s, w2_ref[...], dn,
                        preferred_element_type=jnp.float32) + b2_ref[...]
    h_ref[...] = h
    b = -jnp.sum(h * a_ref[...], axis=1, keepdims=True)
    b_ref[...] = b

    @pl.when(i == 0)
    def _():
        c_ref[0, 0] = -jnp.inf
    c_ref[0, 0] = jnp.maximum(c_ref[0, 0], jnp.max(b))


def _dense(s_pad, w1, b1r, w2, b2r, ar, bs=512):
    nb = N_PAD // bs
    return pl.pallas_call(
        _dense_body,
        grid=(nb,),
        in_specs=[
            pl.BlockSpec((bs, D), lambda i: (i, 0)),
            pl.BlockSpec((D, D), lambda i: (0, 0)),
            pl.BlockSpec((1, D), lambda i: (0, 0)),
            pl.BlockSpec((D, D), lambda i: (0, 0)),
            pl.BlockSpec((1, D), lambda i: (0, 0)),
            pl.BlockSpec((1, D), lambda i: (0, 0)),
        ],
        out_specs=[
            pl.BlockSpec((bs, D), lambda i: (i, 0)),
            pl.BlockSpec((bs, D), lambda i: (i, 0)),
            pl.BlockSpec((bs, 1), lambda i: (i, 0)),
            pl.BlockSpec((1, 1), lambda i: (0, 0),
                         memory_space=pltpu.SMEM),
        ],
        out_shape=[
            jax.ShapeDtypeStruct((N_PAD, D), jnp.float32),
            jax.ShapeDtypeStruct((N_PAD, D), jnp.float32),
            jax.ShapeDtypeStruct((N_PAD, 1), jnp.float32),
            jax.ShapeDtypeStruct((1, 1), jnp.float32),
        ],
    )(s_pad, w1, b1r, w2, b2r, ar)


def _sc_edges_body(ept, h_hbm, src_hbm, dst_hbm,
                   hi_out, hj_out,
                   isrc, idst, hib, hjb,
                   semi0, semj0, semi1, semj1, semi2, semj2):
    cid = lax.axis_index("c")
    sid = lax.axis_index("s")
    gwid = cid * NS + sid
    nch = ept // CH
    base = gwid * ept

    pltpu.sync_copy(src_hbm.at[pl.ds(base, ept)], isrc)
    pltpu.sync_copy(dst_hbm.at[pl.ds(base, ept)], idst)

    sems = ((semi0, semj0), (semi1, semj1), (semi2, semj2))
    nbuf = 3

    def _issue(c, s):
        e0 = c * CH
        si, sj = sems[s]
        pltpu.async_copy(h_hbm.at[idst.at[pl.ds(e0, CH)]], hib.at[s], si)
        pltpu.async_copy(h_hbm.at[isrc.at[pl.ds(e0, CH)]], hjb.at[s], sj)

    def _drain(c, s):
        e0 = c * CH
        si, sj = sems[s]
        pltpu.make_async_copy(h_hbm.at[idst.at[pl.ds(e0, CH)]],
                              hib.at[s], si).wait()
        pltpu.make_async_copy(h_hbm.at[isrc.at[pl.ds(e0, CH)]],
                              hjb.at[s], sj).wait()
        pltpu.sync_copy(hib.at[s], hi_out.at[pl.ds(base + e0, CH)])
        pltpu.sync_copy(hjb.at[s], hj_out.at[pl.ds(base + e0, CH)])

    _issue(0, 0)
    _issue(1, 1)

    def _chunk(c, _):
        for s in range(nbuf):
            @pl.when(c % nbuf == s)
            def _(s=s):
                @pl.when(c + 2 < nch)
                def _():
                    _issue(c + 2, (s + 2) % nbuf)
                _drain(c, s)
        return 0
    lax.fori_loop(0, nch, _chunk, 0)


def _sc_edges(h, src, dst, e):
    ept = e // NW
    mesh = plsc.VectorSubcoreMesh(core_axis_name="c", subcore_axis_name="s",
                                  num_cores=NC, num_subcores=NS)
    f = pl.kernel(
        functools.partial(_sc_edges_body, ept),
        out_type=[
            jax.ShapeDtypeStruct((e, D), jnp.float32),
            jax.ShapeDtypeStruct((e, D), jnp.float32),
        ],
        mesh=mesh,
        compiler_params=pltpu.CompilerParams(needs_layout_passes=False,
                                             use_tc_tiling_on_sc=False),
        scratch_types=[
            pltpu.VMEM((ept,), jnp.int32),
            pltpu.VMEM((ept,), jnp.int32),
            pltpu.VMEM((3, CH, D), jnp.float32),
            pltpu.VMEM((3, CH, D), jnp.float32),
            pltpu.SemaphoreType.DMA,
            pltpu.SemaphoreType.DMA,
            pltpu.SemaphoreType.DMA,
            pltpu.SemaphoreType.DMA,
            pltpu.SemaphoreType.DMA,
            pltpu.SemaphoreType.DMA,
        ],
    )
    return f(h, src, dst)


def _edgew_body(hi_ref, hj_ref, c_ref, a_ref, r_ref):
    hi = hi_ref[...]
    hj = hj_ref[...]
    d = hi - hj
    db = d.astype(jnp.bfloat16).astype(jnp.float32)
    ab = a_ref[...].astype(jnp.bfloat16).astype(jnp.float32)
    e = jnp.sum(db * ab, axis=1, keepdims=True)
    ha = jnp.sum(hi * a_ref[...], axis=1, keepdims=True)
    w = jnp.exp(e - ha - c_ref[0, 0])
    r_ref[:, :D] = hj * w
    r_ref[:, D:] = jnp.concatenate(
        [w, jnp.zeros((w.shape[0], W - D - 1), jnp.float32)], axis=1)


def _edgew(hi, hj, cmax, ar, e, bs=8000):
    nb = e // bs
    return pl.pallas_call(
        _edgew_body,
        grid=(nb,),
        in_specs=[
            pl.BlockSpec((bs, D), lambda i: (i, 0)),
            pl.BlockSpec((bs, D), lambda i: (i, 0)),
            pl.BlockSpec((1, 1), lambda i: (0, 0),
                         memory_space=pltpu.SMEM),
            pl.BlockSpec((1, D), lambda i: (0, 0)),
        ],
        out_specs=pl.BlockSpec((bs, W), lambda i: (i, 0)),
        out_shape=jax.ShapeDtypeStruct((e, W), jnp.float32),
    )(hi, hj, cmax, ar)


def _sc_scatter_body(ept, r_hbm, dst_hbm, zc_hbm,
                     acc_out, idst, idb, rbuf, acc, sem, sem1):
    cid = lax.axis_index("c")
    sid = lax.axis_index("s")
    gwid = cid * NS + sid
    nch = ept // CH
    base = gwid * ept
    row0 = sid * (N_PAD // NS)
    rpt = N_PAD // NS

    pltpu.sync_copy(zc_hbm.at[pl.ds(row0, rpt)], acc.at[pl.ds(row0, rpt)])
    pltpu.sync_copy(dst_hbm.at[pl.ds(base, ept)], idst)
    plsc.subcore_barrier()

    sems = (sem, sem1)

    def _issue(c, s):
        pltpu.async_copy(r_hbm.at[pl.ds(base + c * CH, CH)],
                         rbuf.at[s], sems[s])

    def _drain(c, s):
        e0 = c * CH
        pltpu.make_async_copy(r_hbm.at[pl.ds(base + e0, CH)],
                              rbuf.at[s], sems[s]).wait()
        for k in range(CH // 16):
            idb[s, pl.ds(k * 16, 16)] = idst[pl.ds(e0 + k * 16, 16)]
        pltpu.sync_copy(rbuf.at[s], acc.at[idb.at[s]], add=True)

    _issue(0, 0)

    def _chunk(c, _):
        @pl.when(c % 2 == 0)
        def _():
            @pl.when(c + 1 < nch)
            def _():
                _issue(c + 1, 1)
            _drain(c, 0)

        @pl.when(c % 2 == 1)
        def _():
            @pl.when(c + 1 < nch)
            def _():
                _issue(c + 1, 0)
            _drain(c, 1)
        return 0
    lax.fori_loop(0, nch, _chunk, 0)

    plsc.subcore_barrier()
    pltpu.sync_copy(acc.at[pl.ds(row0, rpt)],
                    acc_out.at[cid, pl.ds(row0, rpt)])


def _sc_scatter(r, dst, zc, e):
    ept = e // NW
    mesh = plsc.VectorSubcoreMesh(core_axis_name="c", subcore_axis_name="s",
                                  num_cores=NC, num_subcores=NS)
    f = pl.kernel(
        functools.partial(_sc_scatter_body, ept),
        out_type=jax.ShapeDtypeStruct((NC, N_PAD, W), jnp.float32),
        mesh=mesh,
        compiler_params=pltpu.CompilerParams(needs_layout_passes=False,
                                             use_tc_tiling_on_sc=False),
        scratch_types=[
            pltpu.VMEM((ept,), jnp.int32),
            pltpu.VMEM((2, CH), jnp.int32),
            pltpu.VMEM((2, CH, W), jnp.float32),
            pltpu.VMEM_SHARED((N_PAD, W), jnp.float32),
            pltpu.SemaphoreType.DMA,
            pltpu.SemaphoreType.DMA,
        ],
    )
    return f(r, dst, zc)


def _merge_body(z_ref, a0_ref, a1_ref, o_ref):
    num = a0_ref[0, :, :D] + a1_ref[0, :, :D]
    den = a0_ref[0, :, D:D + 1] + a1_ref[0, :, D:D + 1]
    o_ref[...] = jnp.maximum(z_ref[...] + num / (den + EPS), 0.0)


def _merge(z, accs, bs=512):
    nb = N_PAD // bs
    return pl.pallas_call(
        _merge_body,
        grid=(nb,),
        in_specs=[
            pl.BlockSpec((bs, D), lambda i: (i, 0)),
            pl.BlockSpec((1, bs, W), lambda i: (0, i, 0)),
            pl.BlockSpec((1, bs, W), lambda i: (1, i, 0)),
        ],
        out_specs=pl.BlockSpec((bs, D), lambda i: (i, 0)),
        out_shape=jax.ShapeDtypeStruct((N_PAD, D), jnp.float32),
    )(z, accs, accs)


def _dense2_body(z_ref, a0_ref, a1_ref, w1_ref, b1_ref, w2_ref, b2_ref,
                 a_ref, zo_ref, h_ref, b_ref, c_ref):
    i = pl.program_id(0)
    num = a0_ref[0, :, :D] + a1_ref[0, :, :D]
    den = a0_ref[0, :, D:D + 1] + a1_ref[0, :, D:D + 1]
    s = jnp.maximum(z_ref[...] + num / (den + EPS), 0.0)
    dn = (((1,), (1,)), ((), ()))
    zo_ref[...] = lax.dot_general(s, w1_ref[...], dn,
                                  preferred_element_type=jnp.float32) + b1_ref[...]
    h = lax.dot_general(s, w2_ref[...], dn,
                        preferred_element_type=jnp.float32) + b2_ref[...]
    h_ref[...] = h
    b = -jnp.sum(h * a_ref[...], axis=1, keepdims=True)
    b_ref[...] = b

    @pl.when(i == 0)
    def _():
        c_ref[0, 0] = -jnp.inf
    c_ref[0, 0] = jnp.maximum(c_ref[0, 0], jnp.max(b))


def _dense2(z, accs, w1, b1r, w2, b2r, ar, bs=512):
    nb = N_PAD // bs
    return pl.pallas_call(
        _dense2_body,
        grid=(nb,),
        in_specs=[
            pl.BlockSpec((bs, D), lambda i: (i, 0)),
            pl.BlockSpec((1, bs, W), lambda i: (0, i, 0)),
            pl.BlockSpec((1, bs, W), lambda i: (1, i, 0)),
            pl.BlockSpec((D, D), lambda i: (0, 0)),
            pl.BlockSpec((1, D), lambda i: (0, 0)),
            pl.BlockSpec((D, D), lambda i: (0, 0)),
            pl.BlockSpec((1, D), lambda i: (0, 0)),
            pl.BlockSpec((1, D), lambda i: (0, 0)),
        ],
        out_specs=[
            pl.BlockSpec((bs, D), lambda i: (i, 0)),
            pl.BlockSpec((bs, D), lambda i: (i, 0)),
            pl.BlockSpec((bs, 1), lambda i: (i, 0)),
            pl.BlockSpec((1, 1), lambda i: (0, 0),
                         memory_space=pltpu.SMEM),
        ],
        out_shape=[
            jax.ShapeDtypeStruct((N_PAD, D), jnp.float32),
            jax.ShapeDtypeStruct((N_PAD, D), jnp.float32),
            jax.ShapeDtypeStruct((N_PAD, 1), jnp.float32),
            jax.ShapeDtypeStruct((1, 1), jnp.float32),
        ],
    )(z, accs, accs, w1, b1r, w2, b2r, ar)


def kernel(s, edge_index,
           w1_0, b1_0, w2_0, b2_0, a_0,
           w1_1, b1_1, w2_1, b2_1, a_1,
           w1_2, b1_2, w2_2, b2_2, a_2):
    n, d = s.shape
    e = edge_index.shape[1]
    assert d == D and e % NW == 0 and (e // NW) % CH == 0

    src = edge_index[0]
    dst = edge_index[1]
    zc = jnp.zeros((N_PAD, W), jnp.float32)
    s_pad = jnp.pad(s, ((0, N_PAD - n), (0, 0)))

    params = [
        (w1_0, b1_0, w2_0, b2_0, a_0),
        (w1_1, b1_1, w2_1, b2_1, a_1),
        (w1_2, b1_2, w2_2, b2_2, a_2),
    ]
    z = accs = None
    for li, (w1, b1, w2, b2, a) in enumerate(params):
        b1r = b1.reshape(1, D)
        b2r = b2.reshape(1, D)
        ar = a.reshape(1, D)
        if li == 0:
            z, h, bneg, cmax = _dense(s_pad, w1, b1r, w2, b2r, ar)
        else:
            z, h, bneg, cmax = _dense2(z, accs, w1, b1r, w2, b2r, ar)
        hi, hj = _sc_edges(h, src, dst, e)
        r = _edgew(hi, hj, cmax, ar, e)
        accs = _sc_scatter(r, dst, zc, e)
    out = _merge(z, accs)
    return out[:n]


# speedup vs baseline: 1.0162x; 1.0162x over previous
import functools

import jax
import jax.numpy as jnp
from jax import lax
from jax.experimental import pallas as pl
from jax.experimental.pallas import tpu as pltpu
from jax.experimental.pallas import tpu_sc as plsc

N_PAD = 10240
D = 128
W = 144
NC = 2
NS = 16
NW = NC * NS
CH = 80
EPS = 1e-30


def _dense_body(s_ref, w1_ref, b1_ref, w2_ref, b2_ref, a_ref,
                z_ref, h_ref, b_ref, c_ref):
    i = pl.program_id(0)
    s = s_ref[...]
    dn = (((1,), (1,)), ((), ()))
    z_ref[...] = lax.dot_general(s, w1_ref[...], dn,
                                 preferred_element_type=jnp.float32) + b1_ref[...]
    h = lax.dot_general(s, w2_ref[...], dn,
                        preferred_element_type=jnp.float32) + b2_ref[...]
    h_ref[...] = h
    b = -jnp.sum(h * a_ref[...], axis=1, keepdims=True)
    b_ref[...] = b

    @pl.when(i == 0)
    def _():
        c_ref[0, 0] = -jnp.inf
    c_ref[0, 0] = jnp.maximum(c_ref[0, 0], jnp.max(b))


def _dense(s_pad, w1, b1r, w2, b2r, ar, bs=512):
    nb = N_PAD // bs
    return pl.pallas_call(
        _dense_body,
        grid=(nb,),
        in_specs=[
            pl.BlockSpec((bs, D), lambda i: (i, 0)),
            pl.BlockSpec((D, D), lambda i: (0, 0)),
            pl.BlockSpec((1, D), lambda i: (0, 0)),
            pl.BlockSpec((D, D), lambda i: (0, 0)),
            pl.BlockSpec((1, D), lambda i: (0, 0)),
            pl.BlockSpec((1, D), lambda i: (0, 0)),
        ],
        out_specs=[
            pl.BlockSpec((bs, D), lambda i: (i, 0)),
            pl.BlockSpec((bs, D), lambda i: (i, 0)),
            pl.BlockSpec((bs, 1), lambda i: (i, 0)),
            pl.BlockSpec((1, 1), lambda i: (0, 0),
                         memory_space=pltpu.SMEM),
        ],
        out_shape=[
            jax.ShapeDtypeStruct((N_PAD, D), jnp.float32),
            jax.ShapeDtypeStruct((N_PAD, D), jnp.float32),
            jax.ShapeDtypeStruct((N_PAD, 1), jnp.float32),
            jax.ShapeDtypeStruct((1, 1), jnp.float32),
        ],
    )(s_pad, w1, b1r, w2, b2r, ar)


def _sc_edges_body(ept, h_hbm, src_hbm, dst_hbm,
                   hi_out, hj_out,
                   isrc, idst, hib, hjb,
                   semi0, semj0, semi1, semj1):
    cid = lax.axis_index("c")
    sid = lax.axis_index("s")
    gwid = cid * NS + sid
    nch = ept // CH
    base = gwid * ept

    pltpu.sync_copy(src_hbm.at[pl.ds(base, ept)], isrc)
    pltpu.sync_copy(dst_hbm.at[pl.ds(base, ept)], idst)

    sems = ((semi0, semj0), (semi1, semj1))
    nbuf = 2

    def _issue(c, s):
        e0 = c * CH
        si, sj = sems[s]
        pltpu.async_copy(h_hbm.at[idst.at[pl.ds(e0, CH)]], hib.at[s], si)
        pltpu.async_copy(h_hbm.at[isrc.at[pl.ds(e0, CH)]], hjb.at[s], sj)

    def _drain(c, s):
        e0 = c * CH
        si, sj = sems[s]
        pltpu.make_async_copy(h_hbm.at[idst.at[pl.ds(e0, CH)]],
                              hib.at[s], si).wait()
        pltpu.make_async_copy(h_hbm.at[isrc.at[pl.ds(e0, CH)]],
                              hjb.at[s], sj).wait()
        pltpu.sync_copy(hib.at[s], hi_out.at[pl.ds(base + e0, CH)])
        pltpu.sync_copy(hjb.at[s], hj_out.at[pl.ds(base + e0, CH)])

    _issue(0, 0)

    def _chunk(c, _):
        for s in range(nbuf):
            @pl.when(c % nbuf == s)
            def _(s=s):
                @pl.when(c + 1 < nch)
                def _():
                    _issue(c + 1, (s + 1) % nbuf)
                _drain(c, s)
        return 0
    lax.fori_loop(0, nch, _chunk, 0)


def _sc_edges(h, src, dst, e):
    ept = e // NW
    mesh = plsc.VectorSubcoreMesh(core_axis_name="c", subcore_axis_name="s",
                                  num_cores=NC, num_subcores=NS)
    f = pl.kernel(
        functools.partial(_sc_edges_body, ept),
        out_type=[
            jax.ShapeDtypeStruct((e, D), jnp.float32),
            jax.ShapeDtypeStruct((e, D), jnp.float32),
        ],
        mesh=mesh,
        compiler_params=pltpu.CompilerParams(needs_layout_passes=False,
                                             use_tc_tiling_on_sc=False),
        scratch_types=[
            pltpu.VMEM((ept,), jnp.int32),
            pltpu.VMEM((ept,), jnp.int32),
            pltpu.VMEM((2, CH, D), jnp.float32),
            pltpu.VMEM((2, CH, D), jnp.float32),
            pltpu.SemaphoreType.DMA,
            pltpu.SemaphoreType.DMA,
            pltpu.SemaphoreType.DMA,
            pltpu.SemaphoreType.DMA,
        ],
    )
    return f(h, src, dst)


def _edgew_body(hi_ref, hj_ref, c_ref, a_ref, r_ref):
    hi = hi_ref[...]
    hj = hj_ref[...]
    d = hi - hj
    db = d.astype(jnp.bfloat16).astype(jnp.float32)
    ab = a_ref[...].astype(jnp.bfloat16).astype(jnp.float32)
    e = jnp.sum(db * ab, axis=1, keepdims=True)
    ha = jnp.sum(hi * a_ref[...], axis=1, keepdims=True)
    w = jnp.exp(e - ha - c_ref[0, 0])
    r_ref[:, :D] = hj * w
    r_ref[:, D:] = jnp.concatenate(
        [w, jnp.zeros((w.shape[0], W - D - 1), jnp.float32)], axis=1)


def _edgew(hi, hj, cmax, ar, e, bs=8000):
    nb = e // bs
    return pl.pallas_call(
        _edgew_body,
        grid=(nb,),
        in_specs=[
            pl.BlockSpec((bs, D), lambda i: (i, 0)),
            pl.BlockSpec((bs, D), lambda i: (i, 0)),
            pl.BlockSpec((1, 1), lambda i: (0, 0),
                         memory_space=pltpu.SMEM),
            pl.BlockSpec((1, D), lambda i: (0, 0)),
        ],
        out_specs=pl.BlockSpec((bs, W), lambda i: (i, 0)),
        out_shape=jax.ShapeDtypeStruct((e, W), jnp.float32),
    )(hi, hj, cmax, ar)


def _sc_scatter_body(ept, r_hbm, dst_hbm, zc_hbm,
                     acc_out, idst, idb, rbuf, acc, sem, sem1):
    cid = lax.axis_index("c")
    sid = lax.axis_index("s")
    gwid = cid * NS + sid
    nch = ept // CH
    base = gwid * ept
    row0 = sid * (N_PAD // NS)
    rpt = N_PAD // NS

    pltpu.sync_copy(zc_hbm.at[pl.ds(row0, rpt)], acc.at[pl.ds(row0, rpt)])
    pltpu.sync_copy(dst_hbm.at[pl.ds(base, ept)], idst)
    plsc.subcore_barrier()

    sems = (sem, sem1)

    def _issue(c, s):
        pltpu.async_copy(r_hbm.at[pl.ds(base + c * CH, CH)],
                         rbuf.at[s], sems[s])

    def _drain(c, s):
        e0 = c * CH
        pltpu.make_async_copy(r_hbm.at[pl.ds(base + e0, CH)],
                              rbuf.at[s], sems[s]).wait()
        for k in range(CH // 16):
            idb[s, pl.ds(k * 16, 16)] = idst[pl.ds(e0 + k * 16, 16)]
        pltpu.sync_copy(rbuf.at[s], acc.at[idb.at[s]], add=True)

    _issue(0, 0)

    def _chunk(c, _):
        @pl.when(c % 2 == 0)
        def _():
            @pl.when(c + 1 < nch)
            def _():
                _issue(c + 1, 1)
            _drain(c, 0)

        @pl.when(c % 2 == 1)
        def _():
            @pl.when(c + 1 < nch)
            def _():
                _issue(c + 1, 0)
            _drain(c, 1)
        return 0
    lax.fori_loop(0, nch, _chunk, 0)

    plsc.subcore_barrier()
    pltpu.sync_copy(acc.at[pl.ds(row0, rpt)],
                    acc_out.at[cid, pl.ds(row0, rpt)])


def _sc_scatter(r, dst, zc, e):
    ept = e // NW
    mesh = plsc.VectorSubcoreMesh(core_axis_name="c", subcore_axis_name="s",
                                  num_cores=NC, num_subcores=NS)
    f = pl.kernel(
        functools.partial(_sc_scatter_body, ept),
        out_type=jax.ShapeDtypeStruct((NC, N_PAD, W), jnp.float32),
        mesh=mesh,
        compiler_params=pltpu.CompilerParams(needs_layout_passes=False,
                                             use_tc_tiling_on_sc=False),
        scratch_types=[
            pltpu.VMEM((ept,), jnp.int32),
            pltpu.VMEM((2, CH), jnp.int32),
            pltpu.VMEM((2, CH, W), jnp.float32),
            pltpu.VMEM_SHARED((N_PAD, W), jnp.float32),
            pltpu.SemaphoreType.DMA,
            pltpu.SemaphoreType.DMA,
        ],
    )
    return f(r, dst, zc)


def _merge_body(z_ref, a0_ref, a1_ref, o_ref):
    num = a0_ref[0, :, :D] + a1_ref[0, :, :D]
    den = a0_ref[0, :, D:D + 1] + a1_ref[0, :, D:D + 1]
    o_ref[...] = jnp.maximum(z_ref[...] + num / (den + EPS), 0.0)


def _merge(z, accs, bs=512):
    nb = N_PAD // bs
    return pl.pallas_call(
        _merge_body,
        grid=(nb,),
        in_specs=[
            pl.BlockSpec((bs, D), lambda i: (i, 0)),
            pl.BlockSpec((1, bs, W), lambda i: (0, i, 0)),
            pl.BlockSpec((1, bs, W), lambda i: (1, i, 0)),
        ],
        out_specs=pl.BlockSpec((bs, D), lambda i: (i, 0)),
        out_shape=jax.ShapeDtypeStruct((N_PAD, D), jnp.float32),
    )(z, accs, accs)


def _dense2_body(z_ref, a0_ref, a1_ref, w1_ref, b1_ref, w2_ref, b2_ref,
                 a_ref, zo_ref, h_ref, b_ref, c_ref):
    i = pl.program_id(0)
    num = a0_ref[0, :, :D] + a1_ref[0, :, :D]
    den = a0_ref[0, :, D:D + 1] + a1_ref[0, :, D:D + 1]
    s = jnp.maximum(z_ref[...] + num / (den + EPS), 0.0)
    dn = (((1,), (1,)), ((), ()))
    zo_ref[...] = lax.dot_general(s, w1_ref[...], dn,
                                  preferred_element_type=jnp.float32) + b1_ref[...]
    h = lax.dot_general(s, w2_ref[...], dn,
                        preferred_element_type=jnp.float32) + b2_ref[...]
    h_ref[...] = h
    b = -jnp.sum(h * a_ref[...], axis=1, keepdims=True)
    b_ref[...] = b

    @pl.when(i == 0)
    def _():
        c_ref[0, 0] = -jnp.inf
    c_ref[0, 0] = jnp.maximum(c_ref[0, 0], jnp.max(b))


def _dense2(z, accs, w1, b1r, w2, b2r, ar, bs=512):
    nb = N_PAD // bs
    return pl.pallas_call(
        _dense2_body,
        grid=(nb,),
        in_specs=[
            pl.BlockSpec((bs, D), lambda i: (i, 0)),
            pl.BlockSpec((1, bs, W), lambda i: (0, i, 0)),
            pl.BlockSpec((1, bs, W), lambda i: (1, i, 0)),
            pl.BlockSpec((D, D), lambda i: (0, 0)),
            pl.BlockSpec((1, D), lambda i: (0, 0)),
            pl.BlockSpec((D, D), lambda i: (0, 0)),
            pl.BlockSpec((1, D), lambda i: (0, 0)),
            pl.BlockSpec((1, D), lambda i: (0, 0)),
        ],
        out_specs=[
            pl.BlockSpec((bs, D), lambda i: (i, 0)),
            pl.BlockSpec((bs, D), lambda i: (i, 0)),
            pl.BlockSpec((bs, 1), lambda i: (i, 0)),
            pl.BlockSpec((1, 1), lambda i: (0, 0),
                         memory_space=pltpu.SMEM),
        ],
        out_shape=[
            jax.ShapeDtypeStruct((N_PAD, D), jnp.float32),
            jax.ShapeDtypeStruct((N_PAD, D), jnp.float32),
            jax.ShapeDtypeStruct((N_PAD, 1), jnp.float32),
            jax.ShapeDtypeStruct((1, 1), jnp.float32),
        ],
    )(z, accs, accs, w1, b1r, w2, b2r, ar)


def kernel(s, edge_index,
           w1_0, b1_0, w2_0, b2_0, a_0,
           w1_1, b1_1, w2_1, b2_1, a_1,
           w1_2, b1_2, w2_2, b2_2, a_2):
    n, d = s.shape
    e = edge_index.shape[1]
    assert d == D and e % NW == 0 and (e // NW) % CH == 0

    src = edge_index[0]
    dst = edge_index[1]
    zc = jnp.zeros((N_PAD, W), jnp.float32)
    s_pad = jnp.pad(s, ((0, N_PAD - n), (0, 0)))

    params = [
        (w1_0, b1_0, w2_0, b2_0, a_0),
        (w1_1, b1_1, w2_1, b2_1, a_1),
        (w1_2, b1_2, w2_2, b2_2, a_2),
    ]
    z = accs = None
    for li, (w1, b1, w2, b2, a) in enumerate(params):
        b1r = b1.reshape(1, D)
        b2r = b2.reshape(1, D)
        ar = a.reshape(1, D)
        if li == 0:
            z, h, bneg, cmax = _dense(s_pad, w1, b1r, w2, b2r, ar)
        else:
            z, h, bneg, cmax = _dense2(z, accs, w1, b1r, w2, b2r, ar)
        hi, hj = _sc_edges(h, src, dst, e)
        r = _edgew(hi, hj, cmax, ar, e)
        accs = _sc_scatter(r, dst, zc, e)
    out = _merge(z, accs)
    return out[:n]
